# Initial kernel scaffold; baseline (speedup 1.0000x reference)
import functools

import jax
import jax.numpy as jnp
from jax import lax
from jax.experimental import pallas as pl
from jax.experimental.pallas import tpu as pltpu
from jax.experimental.pallas import tpu_sc as plsc

NC = 2
NS = 16
LANES = 16
CH = 128
ZR = 80
D = 128


def _make_sc_aggregate(n_pad, e_pad, with_counts):
    nchunk = e_pad // (NC * NS * CH)
    rows_per_sub = n_pad // NS
    mesh = plsc.VectorSubcoreMesh(core_axis_name="c", subcore_axis_name="s")

    out_type = [jax.ShapeDtypeStruct((NC, n_pad, D), jnp.float32)]
    scratch = [
        pltpu.VMEM((CH,), jnp.int32),
        pltpu.VMEM((CH,), jnp.int32),
        pltpu.VMEM((CH, D), jnp.float32),
        pltpu.VMEM((ZR, D), jnp.float32),
        pltpu.VMEM_SHARED((n_pad, D), jnp.float32),
    ]
    if with_counts:
        out_type.append(jax.ShapeDtypeStruct((NC, n_pad, LANES), jnp.float32))
        scratch += [
            pltpu.VMEM((CH, LANES), jnp.float32),
            pltpu.VMEM((ZR, LANES), jnp.float32),
            pltpu.VMEM_SHARED((n_pad, LANES), jnp.float32),
        ]

    @functools.partial(pl.kernel, out_type=tuple(out_type), mesh=mesh,
                       scratch_types=scratch)
    def body(*refs):
        if with_counts:
            (src_hbm, dst_hbm, tab_hbm, sum_out, cnt_out,
             sidx, didx, rows, zbuf, acc, ones, zcbuf, cacc) = refs
        else:
            (src_hbm, dst_hbm, tab_hbm, sum_out,
             sidx, didx, rows, zbuf, acc) = refs

        c = lax.axis_index("c")
        s = lax.axis_index("s")
        zero16 = jnp.zeros((LANES,), jnp.float32)

        @pl.loop(0, ZR)
        def _(i):
            @pl.loop(0, D, step=LANES)
            def _(j):
                zbuf.at[i, pl.ds(j, LANES)][...] = zero16

        if with_counts:
            one16 = jnp.ones((LANES,), jnp.float32)

            @pl.loop(0, ZR)
            def _(i):
                zcbuf.at[i, pl.ds(0, LANES)][...] = zero16

            @pl.loop(0, CH)
            def _(i):
                ones.at[i, pl.ds(0, LANES)][...] = one16

        rbase = s * rows_per_sub

        @pl.loop(0, rows_per_sub, step=ZR)
        def _(r):
            pltpu.sync_copy(zbuf, acc.at[pl.ds(rbase + r, ZR)])

        if with_counts:
            @pl.loop(0, rows_per_sub, step=ZR)
            def _(r):
                pltpu.sync_copy(zcbuf, cacc.at[pl.ds(rbase + r, ZR)])

        plsc.subcore_barrier()

        wid = c * NS + s
        ebase = wid * (nchunk * CH)

        @pl.loop(0, nchunk)
        def _(i):
            off = ebase + i * CH
            pltpu.sync_copy(src_hbm.at[pl.ds(off, CH)], sidx)
            pltpu.sync_copy(dst_hbm.at[pl.ds(off, CH)], didx)
            pltpu.sync_copy(tab_hbm.at[sidx], rows)
            pltpu.sync_copy(rows, acc.at[didx], add=True)
            if with_counts:
                pltpu.sync_copy(ones, cacc.at[didx], add=True)

        plsc.subcore_barrier()

        pltpu.sync_copy(acc.at[pl.ds(rbase, rows_per_sub)],
                        sum_out.at[c, pl.ds(rbase, rows_per_sub)])
        if with_counts:
            pltpu.sync_copy(cacc.at[pl.ds(rbase, rows_per_sub)],
                            cnt_out.at[c, pl.ds(rbase, rows_per_sub)])

    return body


def _make_tc_layer(n_pad, final):
    blk = 1024
    grid = (n_pad // blk,)
    hi = jax.lax.Precision.HIGHEST

    in_specs = [
        pl.BlockSpec((NC, blk, D), lambda i: (0, i, 0)),
        pl.BlockSpec((NC, blk, LANES), lambda i: (0, i, 0)),
        pl.BlockSpec((blk, D), lambda i: (i, 0)),
        pl.BlockSpec((D, D), lambda i: (0, 0)),
        pl.BlockSpec((D, D), lambda i: (0, 0)),
        pl.BlockSpec((1, D), lambda i: (0, 0)),
    ]
    if final:
        in_specs += [
            pl.BlockSpec((D, D), lambda i: (0, 0)),
            pl.BlockSpec((1, D), lambda i: (0, 0)),
        ]

    def body(*refs):
        if final:
            sums, cnts, xr, wl, wr, b, wo, bo, o = refs
        else:
            sums, cnts, xr, wl, wr, b, o = refs
        ssum = sums[0] + sums[1]
        cnt = cnts[0, :, 0:1] + cnts[1, :, 0:1]
        agg = ssum / jnp.maximum(cnt, 1.0)
        h = jnp.dot(agg, wl[...], preferred_element_type=jnp.float32,
                    precision=hi)
        h = h + jnp.dot(xr[...], wr[...], preferred_element_type=jnp.float32,
                        precision=hi)
        h = jnp.maximum(h + b[...], 0.0)
        if final:
            o[...] = jnp.dot(h, wo[...], preferred_element_type=jnp.float32,
                             precision=hi) + bo[...]
        else:
            o[...] = h

    return pl.pallas_call(
        body,
        grid=grid,
        in_specs=in_specs,
        out_specs=pl.BlockSpec((blk, D), lambda i: (i, 0)),
        out_shape=jax.ShapeDtypeStruct((n_pad, D), jnp.float32),
    )


def kernel(x, edge_index, W_l1, W_r1, b1, g1, be1, rm1, rv1,
           W_l2, W_r2, b2, g2, be2, rm2, rv2, W_lin, b_lin):
    n, d = x.shape
    e = edge_index.shape[1]
    out_dim = W_lin.shape[0]

    n_pad = ((n + 1 + 1023) // 1024) * 1024
    chunk = NC * NS * CH
    e_pad = ((e + chunk - 1) // chunk) * chunk

    src = jnp.concatenate(
        [edge_index[0], jnp.zeros((e_pad - e,), jnp.int32)])
    dst = jnp.concatenate(
        [edge_index[1], jnp.full((e_pad - e,), n, jnp.int32)])
    x_pad = jnp.pad(x, ((0, n_pad - n), (0, 0)))

    s1 = g1 * lax.rsqrt(rv1 + 1e-5)
    wl1 = W_l1.T * s1[None, :]
    wr1 = W_r1.T * s1[None, :]
    bb1 = ((b1 - rm1) * s1 + be1)[None, :]
    s2 = g2 * lax.rsqrt(rv2 + 1e-5)
    wl2 = W_l2.T * s2[None, :]
    wr2 = W_r2.T * s2[None, :]
    bb2 = ((b2 - rm2) * s2 + be2)[None, :]
    wlin = jnp.pad(W_lin.T, ((0, 0), (0, D - out_dim)))
    blin = jnp.pad(b_lin, (0, D - out_dim))[None, :]

    sc_agg1 = _make_sc_aggregate(n_pad, e_pad, with_counts=True)
    sum1, cnts = sc_agg1(src, dst, x_pad)
    h1 = _make_tc_layer(n_pad, final=False)(sum1, cnts, x_pad, wl1, wr1, bb1)

    sc_agg2 = _make_sc_aggregate(n_pad, e_pad, with_counts=False)
    (sum2,) = sc_agg2(src, dst, h1)
    out = _make_tc_layer(n_pad, final=True)(
        sum2, cnts, h1, wl2, wr2, bb2, wlin, blin)

    return out[:n, :out_dim]


# speedup vs baseline: 2.9100x; 2.9100x over previous
import functools

import jax
import jax.numpy as jnp
from jax import lax
from jax.experimental import pallas as pl
from jax.experimental.pallas import tpu as pltpu
from jax.experimental.pallas import tpu_sc as plsc

NC = 2
NS = 16
CH = 128
D = 128


def _make_sc_aggregate(n_pad, e_pad):
    nchunk = e_pad // (NC * NS * CH)
    rows_per_sub = n_pad // NS
    mesh = plsc.VectorSubcoreMesh(core_axis_name="c", subcore_axis_name="s")

    out_type = (jax.ShapeDtypeStruct((NC, n_pad, D), jnp.float32),
                jax.ShapeDtypeStruct((NC, n_pad, D), jnp.float32))
    scratch = [
        pltpu.VMEM((CH,), jnp.int32),
        pltpu.VMEM((1, CH), jnp.int32),
        pltpu.VMEM((1, CH), jnp.int32),
        pltpu.VMEM((CH, D), jnp.float32),
        pltpu.VMEM_SHARED((n_pad, D), jnp.float32),
        pltpu.SemaphoreType.DMA,
    ]

    @functools.partial(pl.kernel, out_type=out_type, mesh=mesh,
                       scratch_types=scratch)
    def body(src_hbm, dst_hbm, tab_hbm, zrow_hbm, iota_hbm,
             sum_out, base_out, sidx, didx, iidx, rows, acc, sem):
        c = lax.axis_index("c")
        s = lax.axis_index("s")
        rbase = s * rows_per_sub

        def gather(src, dst):
            pltpu.async_copy(src, dst, sem).wait()

        def scatter(src, dst, add=False):
            pltpu.async_copy(src, dst, sem, add=add).wait()

        pltpu.sync_copy(zrow_hbm, rows)

        @pl.loop(0, rows_per_sub, step=CH)
        def _(r):
            pltpu.sync_copy(iota_hbm.at[pl.ds(rbase + r, CH)], iidx.at[0])
            scatter(rows, acc.at[iidx.at[0]])

        @pl.loop(0, rows_per_sub, step=CH)
        def _(r):
            pltpu.sync_copy(iota_hbm.at[pl.ds(rbase + r, CH)], iidx.at[0])
            gather(acc.at[iidx.at[0]], rows)
            pltpu.sync_copy(rows, base_out.at[c].at[pl.ds(rbase + r, CH)])

        plsc.subcore_barrier()

        wid = c * NS + s
        ebase = wid * (nchunk * CH)

        @pl.loop(0, nchunk)
        def _(i):
            off = ebase + i * CH
            pltpu.sync_copy(src_hbm.at[pl.ds(off, CH)], sidx)
            pltpu.sync_copy(dst_hbm.at[pl.ds(off, CH)], didx.at[0])
            gather(tab_hbm.at[sidx], rows)
            scatter(rows, acc.at[didx.at[0]], add=True)

        plsc.subcore_barrier()

        @pl.loop(0, rows_per_sub, step=CH)
        def _(r):
            pltpu.sync_copy(iota_hbm.at[pl.ds(rbase + r, CH)], iidx.at[0])
            gather(acc.at[iidx.at[0]], rows)
            pltpu.sync_copy(rows, sum_out.at[c].at[pl.ds(rbase + r, CH)])

    return body


def _make_tc_layer(n_pad, final):
    blk = 1024
    grid = (n_pad // blk,)
    hi = jax.lax.Precision.HIGHEST

    in_specs = [
        pl.BlockSpec((NC, blk, D), lambda i: (0, i, 0)),
        pl.BlockSpec((NC, blk, D), lambda i: (0, i, 0)),
        pl.BlockSpec((NC, blk, D), lambda i: (0, i, 0)),
        pl.BlockSpec((NC, blk, D), lambda i: (0, i, 0)),
        pl.BlockSpec((blk, D), lambda i: (i, 0)),
        pl.BlockSpec((D, D), lambda i: (0, 0)),
        pl.BlockSpec((D, D), lambda i: (0, 0)),
        pl.BlockSpec((1, D), lambda i: (0, 0)),
    ]
    if final:
        in_specs += [
            pl.BlockSpec((D, D), lambda i: (0, 0)),
            pl.BlockSpec((1, D), lambda i: (0, 0)),
        ]

    def body(*refs):
        if final:
            sums, bsums, cnts, bcnts, xr, wl, wr, b, wo, bo, o = refs
        else:
            sums, bsums, cnts, bcnts, xr, wl, wr, b, o = refs
        ssum = (sums[0] - bsums[0]) + (sums[1] - bsums[1])
        cnt = ((cnts[0, :, 0:1] - bcnts[0, :, 0:1])
               + (cnts[1, :, 0:1] - bcnts[1, :, 0:1]))
        agg = ssum / jnp.maximum(cnt, 1.0)
        h = jnp.dot(agg, wl[...], preferred_element_type=jnp.float32,
                    precision=hi)
        h = h + jnp.dot(xr[...], wr[...], preferred_element_type=jnp.float32,
                        precision=hi)
        h = jnp.maximum(h + b[...], 0.0)
        if final:
            o[...] = jnp.dot(h, wo[...], preferred_element_type=jnp.float32,
                             precision=hi) + bo[...]
        else:
            o[...] = h

    return pl.pallas_call(
        body,
        grid=grid,
        in_specs=in_specs,
        out_specs=pl.BlockSpec((blk, D), lambda i: (i, 0)),
        out_shape=jax.ShapeDtypeStruct((n_pad, D), jnp.float32),
    )


def kernel(x, edge_index, W_l1, W_r1, b1, g1, be1, rm1, rv1,
           W_l2, W_r2, b2, g2, be2, rm2, rv2, W_lin, b_lin):
    n, d = x.shape
    e = edge_index.shape[1]
    out_dim = W_lin.shape[0]

    n_pad = ((n + 1 + NS * CH - 1) // (NS * CH)) * (NS * CH)
    chunk = NC * NS * CH
    e_pad = ((e + chunk - 1) // chunk) * chunk

    src = jnp.concatenate(
        [edge_index[0], jnp.zeros((e_pad - e,), jnp.int32)])
    dst = jnp.concatenate(
        [edge_index[1], jnp.full((e_pad - e,), n, jnp.int32)])
    x_pad = jnp.pad(x, ((0, n_pad - n), (0, 0)))

    s1 = g1 * lax.rsqrt(rv1 + 1e-5)
    wl1 = W_l1.T * s1[None, :]
    wr1 = W_r1.T * s1[None, :]
    bb1 = ((b1 - rm1) * s1 + be1)[None, :]
    s2 = g2 * lax.rsqrt(rv2 + 1e-5)
    wl2 = W_l2.T * s2[None, :]
    wr2 = W_r2.T * s2[None, :]
    bb2 = ((b2 - rm2) * s2 + be2)[None, :]
    wlin = jnp.pad(W_lin.T, ((0, 0), (0, D - out_dim)))
    blin = jnp.pad(b_lin, (0, D - out_dim))[None, :]

    zrow = jnp.zeros((CH, D), jnp.float32)
    ones_tab = jnp.ones((n_pad, D), jnp.float32)
    iota = jnp.arange(n_pad, dtype=jnp.int32)

    sc_agg = _make_sc_aggregate(n_pad, e_pad)
    sum1, bsum1 = sc_agg(src, dst, x_pad, zrow, iota)
    cnts, bcnts = sc_agg(src, dst, ones_tab, zrow, iota)
    h1 = _make_tc_layer(n_pad, final=False)(
        sum1, bsum1, cnts, bcnts, x_pad, wl1, wr1, bb1)

    sum2, bsum2 = sc_agg(src, dst, h1, zrow, iota)
    out = _make_tc_layer(n_pad, final=True)(
        sum2, bsum2, cnts, bcnts, h1, wl2, wr2, bb2, wlin, blin)

    return out[:n, :out_dim]
